# baseline (speedup 1.0000x reference)
import jax
import jax.numpy as jnp
from jax.experimental import pallas as pl
from jax.experimental.pallas import tpu as pltpu
from jax.experimental.pallas import tpu_sc as plsc

_NC, _NS = 2, 16
_NW = _NC * _NS
_B, _P, _E = 64, 1024, 768
_PW = _P // _NW
_CH = _PW * _E


def _sc_body(x_hbm, pos_hbm, o_hbm, pos_v, xbuf):
    c = jax.lax.axis_index("c")
    s = jax.lax.axis_index("s")
    wid = s * _NC + c
    pltpu.sync_copy(pos_hbm.at[pl.ds(wid * _CH, _CH)], pos_v)

    def batch_body(b, carry):
        off = b * (_P * _E) + wid * _CH
        pltpu.sync_copy(x_hbm.at[pl.ds(off, _CH)], xbuf)

        def add_body(i, carry2):
            sl = pl.ds(i * 16, 16)
            xbuf[sl] = xbuf[sl] + pos_v[sl]
            return carry2

        jax.lax.fori_loop(0, _CH // 16, add_body, 0, unroll=8)
        pltpu.sync_copy(xbuf, o_hbm.at[pl.ds(off, _CH)])
        return carry

    jax.lax.fori_loop(0, _B, batch_body, 0)


def kernel(x, pos_table):
    B, P, E = x.shape
    call = pl.kernel(
        _sc_body,
        out_type=jax.ShapeDtypeStruct((B * P * E,), x.dtype),
        mesh=plsc.VectorSubcoreMesh(core_axis_name="c", subcore_axis_name="s"),
        scratch_types=[
            pltpu.VMEM((_CH,), jnp.float32),
            pltpu.VMEM((_CH,), jnp.float32),
        ],
    )
    out = call(x.reshape(-1), pos_table.reshape(-1))
    return out.reshape(B, P, E)


# speedup vs baseline: 8.4509x; 8.4509x over previous
import jax
import jax.numpy as jnp
from jax.experimental import pallas as pl
from jax.experimental.pallas import tpu as pltpu

_P, _E = 1024, 768
_CH = 2048
_NBUF = 6


def _pipe_kernel(x_hbm, pos_hbm, o_hbm, bufs, pos_v, in_sems, out_sems, pos_sem):
    n_rows = x_hbm.shape[0]
    K = n_rows // _CH

    def in_copy(k, j):
        return pltpu.make_async_copy(
            x_hbm.at[pl.ds(k * _CH, _CH), :], bufs.at[j], in_sems.at[j])

    def out_copy(k, j):
        return pltpu.make_async_copy(
            bufs.at[j], o_hbm.at[pl.ds(k * _CH, _CH), :], out_sems.at[j])

    pltpu.make_async_copy(pos_hbm, pos_v, pos_sem).start()
    for j in range(_NBUF - 1):
        in_copy(j, j).start()
    pltpu.make_async_copy(pos_hbm, pos_v, pos_sem).wait()

    def step(k, carry):
        j = jax.lax.rem(k, _NBUF)
        in_copy(k, j).wait()
        buf = bufs.at[j]
        for t in range(_CH // _P):
            sl = pl.ds(t * _P, _P)
            buf[sl, :] = buf[sl, :] + pos_v[...]
        out_copy(k, j).start()
        kn = k + _NBUF - 1
        jn = jax.lax.rem(kn, _NBUF)

        @pl.when(jnp.logical_and(k >= 1, kn < K))
        def _():
            out_copy(k - 1, jn).wait()

        @pl.when(kn < K)
        def _():
            in_copy(kn, jn).start()

        return carry

    jax.lax.fori_loop(0, K, step, 0)
    for d in range(_NBUF):
        kd = K - _NBUF + d
        out_copy(kd, kd % _NBUF).wait()


def kernel(x, pos_table):
    B, P, E = x.shape
    x2 = x.reshape(B * P, E)
    out = pl.pallas_call(
        _pipe_kernel,
        in_specs=[
            pl.BlockSpec(memory_space=pltpu.HBM),
            pl.BlockSpec(memory_space=pltpu.HBM),
        ],
        out_specs=pl.BlockSpec(memory_space=pltpu.HBM),
        out_shape=jax.ShapeDtypeStruct((B * P, E), x.dtype),
        scratch_shapes=[
            pltpu.VMEM((_NBUF, _CH, _E), jnp.float32),
            pltpu.VMEM((_P, _E), jnp.float32),
            pltpu.SemaphoreType.DMA((_NBUF,)),
            pltpu.SemaphoreType.DMA((_NBUF,)),
            pltpu.SemaphoreType.DMA,
        ],
    )(x2, pos_table)
    return out.reshape(B, P, E)
